# Initial kernel scaffold; baseline (speedup 1.0000x reference)
import jax
import jax.numpy as jnp
from jax.experimental import pallas as pl


def kernel(predictions, targets):
    raise NotImplementedError("write your pallas kernel here")


# speedup vs baseline: 3.1286x; 3.1286x over previous
import functools

import jax
import jax.numpy as jnp
from jax import lax
from jax.experimental import pallas as pl
from jax.experimental.pallas import tpu as pltpu
from jax.experimental.pallas import tpu_sc as plsc

N = 8192
ROWS = 128
NUM_CORES = 2
NUM_SUBCORES = 16
NWORK = NUM_CORES * NUM_SUBCORES
RPW = ROWS // NWORK
NV = N // 16
SHIFTS = (0, 11, 22)
MASKS = (2047, 2047, 1023)
RCHUNKS = 2048 // 16
MU = (N - 1) / 2.0
DEN = float(N) * (float(N) * float(N) - 1.0) / 12.0
MIN_I32 = -(2**31)


def _sc_body(pred_hbm, targ_hbm, out_hbm,
             in_buf, k0, k1, p0, p1, rank, hist, cnt, acc, lossbuf):
    lane = lax.iota(jnp.int32, 16)

    def zero_hist():
        def body(c, _):
            hist[pl.ds(c * 16, 16)] = jnp.zeros(16, jnp.int32)
            return 0
        lax.fori_loop(0, RCHUNKS, body, 0)

    def prefix_hist_to_cnt():
        def body(c, carry):
            v = hist[pl.ds(c * 16, 16)]
            incl = plsc.cumsum(v)
            cnt[pl.ds(c * 16, 16)] = (incl - v) + carry
            return carry + jnp.sum(v)
        lax.fori_loop(0, RCHUNKS, body, jnp.int32(0))

    def digit(key, p):
        return jnp.bitwise_and(jnp.right_shift(key, SHIFTS[p]), MASKS[p])

    def sweep_transform_hist0():
        def body(c, _):
            x = in_buf[pl.ds(c * 16, 16)]
            u = lax.bitcast_convert_type(x, jnp.int32)
            m = jnp.right_shift(u, 31)
            key = jnp.bitwise_xor(u, jnp.bitwise_or(m, MIN_I32))
            k0[pl.ds(c * 16, 16)] = key
            d = digit(key, 0)
            counts, last = plsc.scan_count(d)
            plsc.addupdate_scatter(hist, [d], counts, mask=last)
            return 0
        lax.fori_loop(0, NV, body, 0)

    def sweep_permute(p, ka, pa, kb, pb):
        def body(c, _):
            key = ka[pl.ds(c * 16, 16)]
            d = digit(key, p)
            counts, last = plsc.scan_count(d)
            base = plsc.load_gather(cnt, [d])
            pos = base + counts - 1
            plsc.store_scatter(cnt, [d], pos + 1, mask=last)
            pay = pa[pl.ds(c * 16, 16)] if pa is not None else c * 16 + lane
            plsc.store_scatter(kb, [pos], key)
            plsc.store_scatter(pb, [pos], pay)
            dn = digit(key, p + 1)
            cn, ln = plsc.scan_count(dn)
            plsc.addupdate_scatter(hist, [dn], cn, mask=ln)
            return 0
        lax.fori_loop(0, NV, body, 0)

    def sweep_final(ka, pa, is_target):
        def body(c, _):
            key = ka[pl.ds(c * 16, 16)]
            d = digit(key, 2)
            counts, last = plsc.scan_count(d)
            base = plsc.load_gather(cnt, [d])
            pos = base + counts - 1
            plsc.store_scatter(cnt, [d], pos + 1, mask=last)
            pay = pa[pl.ds(c * 16, 16)]
            if not is_target:
                plsc.store_scatter(rank, [pay], pos)
            else:
                rp = plsc.load_gather(rank, [pay])
                prod = (pos.astype(jnp.float32) - MU) * (
                    rp.astype(jnp.float32) - MU)
                acc[...] += prod
            return 0
        lax.fori_loop(0, NV, body, 0)

    def rank_row(src_hbm, row, is_target):
        pltpu.sync_copy(src_hbm.at[row], in_buf)
        zero_hist()
        sweep_transform_hist0()
        prefix_hist_to_cnt()
        zero_hist()
        sweep_permute(0, k0, None, k1, p1)
        prefix_hist_to_cnt()
        zero_hist()
        sweep_permute(1, k1, p1, k0, p0)
        prefix_hist_to_cnt()
        sweep_final(k0, p0, is_target)

    wid = lax.axis_index("s") * NUM_CORES + lax.axis_index("c")
    loss_vec = jnp.zeros(16, jnp.float32)
    for j in range(RPW):
        row = wid * RPW + j
        rank_row(pred_hbm, row, is_target=False)
        acc[...] = jnp.zeros(16, jnp.float32)
        rank_row(targ_hbm, row, is_target=True)
        s = jnp.sum(acc[...])
        loss_vec = jnp.where(lane == j, 1.0 - s * (1.0 / DEN), loss_vec)
    lossbuf[...] = loss_vec
    pltpu.sync_copy(lossbuf, out_hbm.at[wid])


@jax.jit
def kernel(predictions, targets):
    mesh = plsc.VectorSubcoreMesh(
        core_axis_name="c", subcore_axis_name="s",
        num_cores=NUM_CORES, num_subcores=NUM_SUBCORES)
    run = functools.partial(
        pl.kernel,
        out_type=jax.ShapeDtypeStruct((NWORK, 16), jnp.float32),
        mesh=mesh,
        compiler_params=pltpu.CompilerParams(needs_layout_passes=False),
        scratch_types=[
            pltpu.VMEM((N,), jnp.float32),
            pltpu.VMEM((N,), jnp.int32),
            pltpu.VMEM((N,), jnp.int32),
            pltpu.VMEM((N,), jnp.int32),
            pltpu.VMEM((N,), jnp.int32),
            pltpu.VMEM((N,), jnp.int32),
            pltpu.VMEM((2048,), jnp.int32),
            pltpu.VMEM((2048,), jnp.int32),
            pltpu.VMEM((16,), jnp.float32),
            pltpu.VMEM((16,), jnp.float32),
        ],
    )(_sc_body)
    out = run(predictions, targets)
    return jnp.sum(out) * (1.0 / ROWS)


# speedup vs baseline: 3.3418x; 1.0682x over previous
import functools

import jax
import jax.numpy as jnp
from jax import lax
from jax.experimental import pallas as pl
from jax.experimental.pallas import tpu as pltpu
from jax.experimental.pallas import tpu_sc as plsc

N = 8192
ROWS = 128
NUM_CORES = 2
NUM_SUBCORES = 16
NWORK = NUM_CORES * NUM_SUBCORES
RPW = ROWS // NWORK
NV = N // 16
U = 2
SHIFTS = (0, 11, 22)
MASKS = (2047, 2047, 1023)
RCHUNKS = 2048 // 16
MU = (N - 1) / 2.0
DEN = float(N) * (float(N) * float(N) - 1.0) / 12.0
MIN_I32 = -(2**31)


def _sc_body(pred_hbm, targ_hbm, out_hbm,
             inp_p, inp_t, kp0, kp1, pp0, pp1, kt0, kt1, pt0, pt1,
             rank_p, rank_t, hist_p, cnt_p, hist_t, cnt_t, acc, lossbuf):
    lane = lax.iota(jnp.int32, 16)

    def digit(key, p):
        return jnp.bitwise_and(jnp.right_shift(key, SHIFTS[p]), MASKS[p])

    def zero_hists():
        def body(c, _):
            hist_p[pl.ds(c * 16, 16)] = jnp.zeros(16, jnp.int32)
            hist_t[pl.ds(c * 16, 16)] = jnp.zeros(16, jnp.int32)
            return 0
        lax.fori_loop(0, RCHUNKS, body, 0)

    def prefix_dual():
        def body(c, carry):
            cp, ct = carry
            vp = hist_p[pl.ds(c * 16, 16)]
            vt = hist_t[pl.ds(c * 16, 16)]
            cnt_p[pl.ds(c * 16, 16)] = (plsc.cumsum(vp) - vp) + cp
            cnt_t[pl.ds(c * 16, 16)] = (plsc.cumsum(vt) - vt) + ct
            hist_p[pl.ds(c * 16, 16)] = jnp.zeros(16, jnp.int32)
            hist_t[pl.ds(c * 16, 16)] = jnp.zeros(16, jnp.int32)
            return cp + jnp.sum(vp), ct + jnp.sum(vt)
        lax.fori_loop(0, RCHUNKS, body, (jnp.int32(0), jnp.int32(0)))

    def sweep0_dual():
        def one(inp, kdst, hist, off):
            x = inp[pl.ds(off, 16)]
            u = lax.bitcast_convert_type(x, jnp.int32)
            m = jnp.right_shift(u, 31)
            key = jnp.bitwise_xor(u, jnp.bitwise_or(m, MIN_I32))
            kdst[pl.ds(off, 16)] = key
            d = digit(key, 0)
            counts, last = plsc.scan_count(d)
            plsc.addupdate_scatter(hist, [d], counts, mask=last)

        def body(c, _):
            for uu in range(U):
                off = c * (U * 16) + uu * 16
                one(inp_p, kp0, hist_p, off)
                one(inp_t, kt0, hist_t, off)
            return 0
        lax.fori_loop(0, NV // U, body, 0)

    def permute_dual(p, srcs):
        def one(ka, pa, kb, pb, hist, cnt, off):
            key = ka[pl.ds(off, 16)]
            d = digit(key, p)
            counts, last = plsc.scan_count(d)
            base = plsc.load_gather(cnt, [d])
            pos = base + counts - 1
            plsc.store_scatter(cnt, [d], pos + 1, mask=last)
            pay = pa[pl.ds(off, 16)] if pa is not None else off + lane
            plsc.store_scatter(kb, [pos], key)
            plsc.store_scatter(pb, [pos], pay)
            dn = digit(key, p + 1)
            cn, ln = plsc.scan_count(dn)
            plsc.addupdate_scatter(hist, [dn], cn, mask=ln)

        def body(c, _):
            for uu in range(U):
                off = c * (U * 16) + uu * 16
                for s in srcs:
                    one(*s, off)
            return 0
        lax.fori_loop(0, NV // U, body, 0)

    def final_dual():
        def one(ka, pa, cnt, rank, off):
            key = ka[pl.ds(off, 16)]
            d = digit(key, 2)
            counts, last = plsc.scan_count(d)
            base = plsc.load_gather(cnt, [d])
            pos = base + counts - 1
            plsc.store_scatter(cnt, [d], pos + 1, mask=last)
            pay = pa[pl.ds(off, 16)]
            plsc.store_scatter(rank, [pay], pos)

        def body(c, _):
            for uu in range(U):
                off = c * (U * 16) + uu * 16
                one(kp0, pp0, cnt_p, rank_p, off)
                one(kt0, pt0, cnt_t, rank_t, off)
            return 0
        lax.fori_loop(0, NV // U, body, 0)

    def dot_sweep():
        def body(c, _):
            for uu in range(4):
                off = c * 64 + uu * 16
                rp = rank_p[pl.ds(off, 16)].astype(jnp.float32)
                rt = rank_t[pl.ds(off, 16)].astype(jnp.float32)
                acc[...] += (rp - MU) * (rt - MU)
            return 0
        lax.fori_loop(0, NV // 4, body, 0)

    wid = lax.axis_index("s") * NUM_CORES + lax.axis_index("c")
    loss_vec = jnp.zeros(16, jnp.float32)
    zero_hists()
    for j in range(RPW):
        row = wid * RPW + j
        pltpu.sync_copy(pred_hbm.at[row], inp_p)
        pltpu.sync_copy(targ_hbm.at[row], inp_t)
        sweep0_dual()
        prefix_dual()
        permute_dual(0, ((kp0, None, kp1, pp1, hist_p, cnt_p),
                         (kt0, None, kt1, pt1, hist_t, cnt_t)))
        prefix_dual()
        permute_dual(1, ((kp1, pp1, kp0, pp0, hist_p, cnt_p),
                         (kt1, pt1, kt0, pt0, hist_t, cnt_t)))
        prefix_dual()
        final_dual()
        acc[...] = jnp.zeros(16, jnp.float32)
        dot_sweep()
        s = jnp.sum(acc[...])
        loss_vec = jnp.where(lane == j, 1.0 - s * (1.0 / DEN), loss_vec)
    lossbuf[...] = loss_vec
    pltpu.sync_copy(lossbuf, out_hbm.at[wid])


@jax.jit
def kernel(predictions, targets):
    mesh = plsc.VectorSubcoreMesh(
        core_axis_name="c", subcore_axis_name="s",
        num_cores=NUM_CORES, num_subcores=NUM_SUBCORES)
    run = functools.partial(
        pl.kernel,
        out_type=jax.ShapeDtypeStruct((NWORK, 16), jnp.float32),
        mesh=mesh,
        compiler_params=pltpu.CompilerParams(needs_layout_passes=False),
        scratch_types=[
            pltpu.VMEM((N,), jnp.float32),
            pltpu.VMEM((N,), jnp.float32),
            pltpu.VMEM((N,), jnp.int32),
            pltpu.VMEM((N,), jnp.int32),
            pltpu.VMEM((N,), jnp.int32),
            pltpu.VMEM((N,), jnp.int32),
            pltpu.VMEM((N,), jnp.int32),
            pltpu.VMEM((N,), jnp.int32),
            pltpu.VMEM((N,), jnp.int32),
            pltpu.VMEM((N,), jnp.int32),
            pltpu.VMEM((N,), jnp.int32),
            pltpu.VMEM((N,), jnp.int32),
            pltpu.VMEM((2048,), jnp.int32),
            pltpu.VMEM((2048,), jnp.int32),
            pltpu.VMEM((2048,), jnp.int32),
            pltpu.VMEM((2048,), jnp.int32),
            pltpu.VMEM((16,), jnp.float32),
            pltpu.VMEM((16,), jnp.float32),
        ],
    )(_sc_body)
    out = run(predictions, targets)
    return jnp.sum(out) * (1.0 / ROWS)


# speedup vs baseline: 7.6112x; 2.2776x over previous
import functools

import jax
import jax.numpy as jnp
from jax import lax
from jax.experimental import pallas as pl
from jax.experimental.pallas import tpu as pltpu
from jax.experimental.pallas import tpu_sc as plsc

N = 8192
ROWS = 128
NUM_CORES = 2
NUM_SUBCORES = 16
NWORK = NUM_CORES * NUM_SUBCORES
RPW = ROWS // NWORK
NV = N // 16
U = 2
SHIFTS = (0, 11, 22)
MASKS = (2047, 2047, 1023)
RCHUNKS = 2048 // 16
MU = (N - 1) / 2.0
DEN = float(N) * (float(N) * float(N) - 1.0) / 12.0
MIN_I32 = -(2**31)


def _sc_body(pred_hbm, targ_hbm, out_hbm,
             kp0, kp1, pp0, pp1, kt0, kt1, pt0, pt1,
             rank_p, rank_t, hist_p, cnt_p, hist_t, cnt_t, lossbuf):
    lane = lax.iota(jnp.int32, 16)

    def digit(key, p):
        return jnp.bitwise_and(jnp.right_shift(key, SHIFTS[p]), MASKS[p])

    def zero_hists():
        def body(c, _):
            hist_p[pl.ds(c * 16, 16)] = jnp.zeros(16, jnp.int32)
            hist_t[pl.ds(c * 16, 16)] = jnp.zeros(16, jnp.int32)
            return 0
        lax.fori_loop(0, RCHUNKS, body, 0)

    def prefix_dual():
        def body(c, carry):
            cp, ct = carry
            vp = hist_p[pl.ds(c * 16, 16)]
            vt = hist_t[pl.ds(c * 16, 16)]
            cnt_p[pl.ds(c * 16, 16)] = (plsc.cumsum(vp) - vp) + cp
            cnt_t[pl.ds(c * 16, 16)] = (plsc.cumsum(vt) - vt) + ct
            hist_p[pl.ds(c * 16, 16)] = jnp.zeros(16, jnp.int32)
            hist_t[pl.ds(c * 16, 16)] = jnp.zeros(16, jnp.int32)
            return cp + jnp.sum(vp), ct + jnp.sum(vt)
        lax.fori_loop(0, RCHUNKS, body, (jnp.int32(0), jnp.int32(0)))

    def sweep0():
        @plsc.parallel_loop(0, N, 16, unroll=4)
        def _(off):
            for kref, hist in ((kp0, hist_p), (kt0, hist_t)):
                u = kref[pl.ds(off, 16)]
                m = jnp.right_shift(u, 31)
                key = jnp.bitwise_xor(u, jnp.bitwise_or(m, MIN_I32))
                kref[pl.ds(off, 16)] = key
                d = digit(key, 0)
                counts, last = plsc.scan_count(d)
                plsc.addupdate_scatter(hist, [d], counts, mask=last)

    def permute_dual(p, srcs, final):
        def body(c, _):
            work = []
            for uu in range(U):
                off = c * (U * 16) + uu * 16
                for (ka, pa, kb, pb, hist, cnt) in srcs:
                    key = ka[pl.ds(off, 16)]
                    d = digit(key, p)
                    counts, last = plsc.scan_count(d)
                    pay = pa[pl.ds(off, 16)] if pa is not None else off + lane
                    if not final:
                        dn = digit(key, p + 1)
                        cn, ln = plsc.scan_count(dn)
                    else:
                        dn = cn = ln = None
                    work.append((key, d, counts, last, pay, dn, cn, ln,
                                 kb, pb, hist, cnt))
            poss = []
            for (key, d, counts, last, pay, dn, cn, ln,
                 kb, pb, hist, cnt) in work:
                base = plsc.load_gather(cnt, [d])
                pos = base + counts - 1
                plsc.store_scatter(cnt, [d], pos + 1, mask=last)
                poss.append(pos)
            for pos, (key, d, counts, last, pay, dn, cn, ln,
                      kb, pb, hist, cnt) in zip(poss, work):
                if final:
                    plsc.store_scatter(kb, [pay], pos)
                else:
                    plsc.store_scatter(kb, [pos], key)
                    plsc.store_scatter(pb, [pos], pay)
                    plsc.addupdate_scatter(hist, [dn], cn, mask=ln)
            return 0
        lax.fori_loop(0, NV // U, body, 0)

    def dot_sweep():
        @plsc.parallel_loop(0, N, 16, unroll=4, carry=jnp.zeros(16, jnp.float32))
        def acc(off, a):
            rp = rank_p[pl.ds(off, 16)].astype(jnp.float32)
            rt = rank_t[pl.ds(off, 16)].astype(jnp.float32)
            return a + (rp - MU) * (rt - MU)
        return jnp.sum(acc)

    wid = lax.axis_index("s") * NUM_CORES + lax.axis_index("c")
    loss_vec = jnp.zeros(16, jnp.float32)
    zero_hists()
    for j in range(RPW):
        row = wid * RPW + j
        pltpu.sync_copy(pred_hbm.at[row], kp0)
        pltpu.sync_copy(targ_hbm.at[row], kt0)
        sweep0()
        prefix_dual()
        permute_dual(0, ((kp0, None, kp1, pp1, hist_p, cnt_p),
                         (kt0, None, kt1, pt1, hist_t, cnt_t)), final=False)
        prefix_dual()
        permute_dual(1, ((kp1, pp1, kp0, pp0, hist_p, cnt_p),
                         (kt1, pt1, kt0, pt0, hist_t, cnt_t)), final=False)
        prefix_dual()
        permute_dual(2, ((kp0, pp0, rank_p, None, hist_p, cnt_p),
                         (kt0, pt0, rank_t, None, hist_t, cnt_t)), final=True)
        s = dot_sweep()
        loss_vec = jnp.where(lane == j, 1.0 - s * (1.0 / DEN), loss_vec)
    lossbuf[...] = loss_vec
    pltpu.sync_copy(lossbuf, out_hbm.at[wid])


@jax.jit
def kernel(predictions, targets):
    mesh = plsc.VectorSubcoreMesh(
        core_axis_name="c", subcore_axis_name="s",
        num_cores=NUM_CORES, num_subcores=NUM_SUBCORES)
    run = functools.partial(
        pl.kernel,
        out_type=jax.ShapeDtypeStruct((NWORK, 16), jnp.float32),
        mesh=mesh,
        compiler_params=pltpu.CompilerParams(needs_layout_passes=False),
        scratch_types=[
            pltpu.VMEM((N,), jnp.int32),
            pltpu.VMEM((N,), jnp.int32),
            pltpu.VMEM((N,), jnp.int32),
            pltpu.VMEM((N,), jnp.int32),
            pltpu.VMEM((N,), jnp.int32),
            pltpu.VMEM((N,), jnp.int32),
            pltpu.VMEM((N,), jnp.int32),
            pltpu.VMEM((N,), jnp.int32),
            pltpu.VMEM((N,), jnp.int32),
            pltpu.VMEM((N,), jnp.int32),
            pltpu.VMEM((2048,), jnp.int32),
            pltpu.VMEM((2048,), jnp.int32),
            pltpu.VMEM((2048,), jnp.int32),
            pltpu.VMEM((2048,), jnp.int32),
            pltpu.VMEM((16,), jnp.float32),
        ],
    )(_sc_body)
    pred_bits = lax.bitcast_convert_type(predictions, jnp.int32)
    targ_bits = lax.bitcast_convert_type(targets, jnp.int32)
    out = run(pred_bits, targ_bits)
    return jnp.sum(out) * (1.0 / ROWS)


# speedup vs baseline: 7.8300x; 1.0287x over previous
import jax
import jax.numpy as jnp
from jax import lax
from jax.experimental import pallas as pl
from jax.experimental.pallas import tpu as pltpu
from jax.experimental.pallas import tpu_sc as plsc
import functools

N = 8192
ROWS = 128
NUM_CORES = 2
NUM_SUBCORES = 16
NWORK = NUM_CORES * NUM_SUBCORES
RPW = ROWS // NWORK
Q = 4
QSIZE = N // Q
QV = QSIZE // 16
R = 2048
SHIFTS = (0, 11, 22)
MASKS = (2047, 2047, 1023)
MU = (N - 1) / 2.0
DEN = float(N) * (float(N) * float(N) - 1.0) / 12.0
MIN_I32 = -(2**31)


def _sc_body(pred_hbm, targ_hbm, out_hbm,
             kp, kt, pp0, pp1, pt0, pt1, rank_p, rank_t,
             hist_p, cnt_p, hist_t, cnt_t, lossbuf):
    lane = lax.iota(jnp.int32, 16)

    def digit(key, p):
        return jnp.bitwise_and(jnp.right_shift(key, SHIFTS[p]), MASKS[p])

    def zero_hists():
        def body(c, _):
            hist_p[pl.ds(c * 16, 16)] = jnp.zeros(16, jnp.int32)
            hist_t[pl.ds(c * 16, 16)] = jnp.zeros(16, jnp.int32)
            return 0
        lax.fori_loop(0, (Q * R) // 16, body, 0)

    def sweep0():
        @plsc.parallel_loop(0, QSIZE, 16, unroll=2)
        def _(off):
            for kref, hist in ((kp, hist_p), (kt, hist_t)):
                for q in range(Q):
                    o = q * QSIZE + off
                    u = kref[pl.ds(o, 16)]
                    m = jnp.right_shift(u, 31)
                    key = jnp.bitwise_xor(u, jnp.bitwise_or(m, MIN_I32))
                    kref[pl.ds(o, 16)] = key
                    d = digit(key, 0)
                    counts, last = plsc.scan_count(d)
                    plsc.addupdate_scatter(
                        hist.at[pl.ds(q * R, R)], [d], counts, mask=last)

    def prefix_dual():
        def body(c, carry):
            cp, ct = carry
            off = c * 16
            zeros = jnp.zeros(16, jnp.int32)

            def one(hist, cnt, carry_s):
                hs = [hist[pl.ds(q * R + off, 16)] for q in range(Q)]
                tot = hs[0] + hs[1] + hs[2] + hs[3]
                base = (plsc.cumsum(tot) - tot) + carry_s
                for q in range(Q):
                    cnt[pl.ds(q * R + off, 16)] = base
                    hist[pl.ds(q * R + off, 16)] = zeros
                    if q + 1 < Q:
                        base = base + hs[q]
                return carry_s + jnp.sum(tot)

            return one(hist_p, cnt_p, cp), one(hist_t, cnt_t, ct)
        lax.fori_loop(0, QV, body, (jnp.int32(0), jnp.int32(0)))

    def permute(p, srcs, final):
        def body(c, _):
            front = []
            for (kref, pa, pb, hist, cnt) in srcs:
                for q in range(Q):
                    off = q * QSIZE + c * 16
                    if pa is None:
                        pay = off + lane
                        key = kref[pl.ds(off, 16)]
                    else:
                        pay = pa[pl.ds(off, 16)]
                        key = plsc.load_gather(kref, [pay])
                    d = digit(key, p)
                    counts, last = plsc.scan_count(d)
                    front.append((q, pay, key, d, counts, last,
                                  pb, hist, cnt))
            poss = []
            for (q, pay, key, d, counts, last, pb, hist, cnt) in front:
                cs = cnt.at[pl.ds(q * R, R)]
                base = plsc.load_gather(cs, [d])
                pos = base + counts - 1
                plsc.store_scatter(cs, [d], pos + 1, mask=last)
                poss.append(pos)
            for pos, (q, pay, key, d, counts, last, pb, hist, cnt) in zip(
                    poss, front):
                if final:
                    plsc.store_scatter(pb, [pay], pos)
                else:
                    plsc.store_scatter(pb, [pos], pay)
                    dn = digit(key, p + 1)
                    idx = jnp.bitwise_or(jnp.bitwise_and(pos, -R), dn)
                    cn, ln = plsc.scan_count(idx)
                    plsc.addupdate_scatter(hist, [idx], cn, mask=ln)
            return 0
        lax.fori_loop(0, QV, body, 0)

    def dot_sweep():
        @plsc.parallel_loop(0, N, 16, unroll=4,
                            carry=jnp.zeros(16, jnp.float32))
        def acc(off, a):
            rp = rank_p[pl.ds(off, 16)].astype(jnp.float32)
            rt = rank_t[pl.ds(off, 16)].astype(jnp.float32)
            return a + (rp - MU) * (rt - MU)
        return jnp.sum(acc)

    wid = lax.axis_index("s") * NUM_CORES + lax.axis_index("c")
    zero_hists()

    def row_body(j, loss_vec):
        row = wid * RPW + j
        pltpu.sync_copy(pred_hbm.at[row], kp)
        pltpu.sync_copy(targ_hbm.at[row], kt)
        sweep0()
        prefix_dual()
        permute(0, ((kp, None, pp1, hist_p, cnt_p),
                    (kt, None, pt1, hist_t, cnt_t)), final=False)
        prefix_dual()
        permute(1, ((kp, pp1, pp0, hist_p, cnt_p),
                    (kt, pt1, pt0, hist_t, cnt_t)), final=False)
        prefix_dual()
        permute(2, ((kp, pp0, rank_p, hist_p, cnt_p),
                    (kt, pt0, rank_t, hist_t, cnt_t)), final=True)
        s = dot_sweep()
        return jnp.where(lane == j, 1.0 - s * (1.0 / DEN), loss_vec)

    loss_vec = lax.fori_loop(0, RPW, row_body, jnp.zeros(16, jnp.float32))
    lossbuf[...] = loss_vec
    pltpu.sync_copy(lossbuf, out_hbm.at[wid])


@jax.jit
def kernel(predictions, targets):
    mesh = plsc.VectorSubcoreMesh(
        core_axis_name="c", subcore_axis_name="s",
        num_cores=NUM_CORES, num_subcores=NUM_SUBCORES)
    run = functools.partial(
        pl.kernel,
        out_type=jax.ShapeDtypeStruct((NWORK, 16), jnp.float32),
        mesh=mesh,
        compiler_params=pltpu.CompilerParams(needs_layout_passes=False),
        scratch_types=[
            pltpu.VMEM((N,), jnp.int32),
            pltpu.VMEM((N,), jnp.int32),
            pltpu.VMEM((N,), jnp.int32),
            pltpu.VMEM((N,), jnp.int32),
            pltpu.VMEM((N,), jnp.int32),
            pltpu.VMEM((N,), jnp.int32),
            pltpu.VMEM((N,), jnp.int32),
            pltpu.VMEM((N,), jnp.int32),
            pltpu.VMEM((Q * R,), jnp.int32),
            pltpu.VMEM((Q * R,), jnp.int32),
            pltpu.VMEM((Q * R,), jnp.int32),
            pltpu.VMEM((Q * R,), jnp.int32),
            pltpu.VMEM((16,), jnp.float32),
        ],
    )(_sc_body)
    pred_bits = lax.bitcast_convert_type(predictions, jnp.int32)
    targ_bits = lax.bitcast_convert_type(targets, jnp.int32)
    out = run(pred_bits, targ_bits)
    return jnp.sum(out) * (1.0 / ROWS)
